# baseline (speedup 1.0000x reference)
import functools

import jax
import jax.numpy as jnp
from jax import lax
from jax.experimental import pallas as pl
from jax.experimental.pallas import tpu as pltpu
from jax.experimental.pallas import tpu_sc as plsc

B = 16384
NUM_F = 13
N_CAT = 26
VOCAB = 100000
D = 128
D_HID = 64
N_BLOCKS = 4

NC = 2
NS = 16
NW = NC * NS
ROWS_PER_W = B // NW
SAMP_PER_CHUNK = 4
IDX_PER_CHUNK = SAMP_PER_CHUNK * N_CAT
NCHUNK = ROWS_PER_W // SAMP_PER_CHUNK
NBUF = 8

@functools.cache
def _make_emb_sum_sc():
    mesh = plsc.VectorSubcoreMesh(core_axis_name="c", subcore_axis_name="s")
    return pl.kernel(
        _emb_sum_body,
        mesh=mesh,
        out_type=jax.ShapeDtypeStruct((B, D), jnp.float32),
        scratch_types=[
            pltpu.VMEM((NCHUNK, IDX_PER_CHUNK), jnp.int32),
            pltpu.VMEM((NBUF, IDX_PER_CHUNK, D), jnp.float32),
            pltpu.VMEM((2, SAMP_PER_CHUNK, D), jnp.float32),
            pltpu.SemaphoreType.DMA((NBUF,)),
            pltpu.SemaphoreType.DMA((2,)),
        ],
    )


def _emb_sum_body(table_hbm, idx_hbm, out_hbm, idx_v, rows_v, ostage_v, sems,
                  osems):
    wid = lax.axis_index("s") * NC + lax.axis_index("c")
    pltpu.sync_copy(idx_hbm.at[wid], idx_v)

    def _gather(c, buf):
        return pltpu.make_async_copy(
            table_hbm.at[idx_v.at[c]], rows_v.at[buf], sems.at[buf])

    def _ocopy(c, ob):
        dst = out_hbm.at[pl.ds(wid * ROWS_PER_W + c * SAMP_PER_CHUNK,
                               SAMP_PER_CHUNK)]
        return pltpu.make_async_copy(ostage_v.at[ob], dst, osems.at[ob])

    def _chunk(c, buf, ob):
        @pl.when(c + (NBUF - 1) < NCHUNK)
        def _():
            _gather(c + (NBUF - 1), (buf + NBUF - 1) % NBUF).start()

        _gather(c, buf).wait()

        @pl.when(c >= 2)
        def _():
            _ocopy(c, ob).wait()

        zero = jnp.zeros((16,), jnp.float32)
        for half in range(SAMP_PER_CHUNK // 2):
            def _jbody(jj, accs, half=half):
                out = list(accs)
                for u in range(2):
                    j = jj * 2 + u
                    for s2 in range(2):
                        s = half * 2 + s2
                        for k in range(D // 16):
                            i = s2 * (D // 16) + k
                            out[i] = out[i] + rows_v[buf, s * N_CAT + j,
                                                     pl.ds(k * 16, 16)]
                return tuple(out)

            accs = lax.fori_loop(0, N_CAT // 2, _jbody,
                                 (zero,) * (2 * (D // 16)))
            for s2 in range(2):
                for k in range(D // 16):
                    ostage_v[ob, half * 2 + s2, pl.ds(k * 16, 16)] = (
                        accs[s2 * (D // 16) + k])
        _ocopy(c, ob).start()

    for c in range(NBUF - 1):
        _gather(c, c).start()

    def _body(t, carry):
        c0 = t * NBUF
        for b in range(NBUF):
            _chunk(c0 + b, b, b % 2)
        return carry

    lax.fori_loop(0, NCHUNK // NBUF, _body, 0)
    _ocopy(NCHUNK - 2, 0).wait()
    _ocopy(NCHUNK - 1, 1).wait()


def _mlp_body(xn_ref, emb_ref, Wn_ref, bn_ref, W1_ref, b1_ref, W2_ref,
              b2_ref, Wo_ref, bo_ref, out_ref):
    x = (emb_ref[...]
         + jnp.dot(xn_ref[...], Wn_ref[...], preferred_element_type=jnp.float32)
         + bn_ref[...][None, :])
    for i in range(N_BLOCKS):
        y = jnp.maximum(
            jnp.dot(x, W1_ref[i], preferred_element_type=jnp.float32)
            + b1_ref[i][None, :], 0.0)
        y = jnp.maximum(
            jnp.dot(y, W2_ref[i], preferred_element_type=jnp.float32)
            + b2_ref[i][None, :], 0.0)
        x = x + y
    out_ref[...] = (jnp.dot(x, Wo_ref[...], preferred_element_type=jnp.float32)
                    + bo_ref[...][None, :])


def _mlp_call(xn, emb, Wn, bn, W1, b1, W2, b2, Wo, bo):
    BM = 2048
    grid = (B // BM,)
    row_spec = lambda w: pl.BlockSpec((BM, w), lambda i: (i, 0))
    full = lambda shape: pl.BlockSpec(shape, lambda i: tuple(0 for _ in shape))
    return pl.pallas_call(
        _mlp_body,
        grid=grid,
        in_specs=[
            row_spec(16), row_spec(D),
            full((16, D)), full((D,)),
            full((N_BLOCKS, D, D_HID)), full((N_BLOCKS, D_HID)),
            full((N_BLOCKS, D_HID, D)), full((N_BLOCKS, D)),
            full((D, 1)), full((1,)),
        ],
        out_specs=row_spec(1),
        out_shape=jax.ShapeDtypeStruct((B, 1), jnp.float32),
    )(xn, emb, Wn, bn, W1, b1, W2, b2, Wo, bo)


def kernel(x_num, x_cat, W_num, b_num, tables, W1, b1, W2, b2, W_out, b_out):
    table_flat = tables.reshape(N_CAT * VOCAB, D)
    offs = (jnp.arange(N_CAT, dtype=jnp.int32) * VOCAB)[None, :]
    idx = (x_cat + offs).reshape(NW, NCHUNK, IDX_PER_CHUNK)
    emb = _make_emb_sum_sc()(table_flat, idx)
    xn = jnp.pad(x_num, ((0, 0), (0, 16 - NUM_F)))
    Wn = jnp.pad(W_num, ((0, 16 - NUM_F), (0, 0)))
    return _mlp_call(xn, emb, Wn, b_num, W1, b1, W2, b2, W_out, b_out)


# speedup vs baseline: 1.0485x; 1.0485x over previous
import functools

import jax
import jax.numpy as jnp
from jax import lax
from jax.experimental import pallas as pl
from jax.experimental.pallas import tpu as pltpu
from jax.experimental.pallas import tpu_sc as plsc

B = 16384
NUM_F = 13
N_CAT = 26
VOCAB = 100000
D = 128
D_HID = 64
N_BLOCKS = 4

NC = 2
NS = 16
NW = NC * NS
SAMP_PER_CHUNK = 4
IDX_PER_CHUNK = SAMP_PER_CHUNK * N_CAT
NBUF = 4


@functools.cache
def _make_emb_sum_sc(nchunk):
    rows_per_w = nchunk * SAMP_PER_CHUNK
    nb = NW * rows_per_w

    def _emb_sum_body(table_hbm, idx_hbm, out_hbm, idx_v, rows_v, ostage_v,
                      sems, osems):
        wid = lax.axis_index("s") * NC + lax.axis_index("c")
        pltpu.sync_copy(idx_hbm.at[wid], idx_v)

        def _gather(c, buf):
            return pltpu.make_async_copy(
                table_hbm.at[idx_v.at[c]], rows_v.at[buf], sems.at[buf])

        def _ocopy(c, ob):
            dst = out_hbm.at[pl.ds(wid * rows_per_w + c * SAMP_PER_CHUNK,
                                   SAMP_PER_CHUNK)]
            return pltpu.make_async_copy(ostage_v.at[ob], dst, osems.at[ob])

        def _chunk(c, buf, ob):
            @pl.when(c + (NBUF - 1) < nchunk)
            def _():
                _gather(c + (NBUF - 1), (buf + NBUF - 1) % NBUF).start()

            _gather(c, buf).wait()

            @pl.when(c >= 2)
            def _():
                _ocopy(c, ob).wait()

            zero = jnp.zeros((16,), jnp.float32)
            for half in range(SAMP_PER_CHUNK // 2):
                def _jbody(jj, accs, half=half):
                    out = list(accs)
                    for u in range(2):
                        j = jj * 2 + u
                        for s2 in range(2):
                            s = half * 2 + s2
                            for k in range(D // 16):
                                i = s2 * (D // 16) + k
                                out[i] = out[i] + rows_v[buf, s * N_CAT + j,
                                                         pl.ds(k * 16, 16)]
                    return tuple(out)

                accs = lax.fori_loop(0, N_CAT // 2, _jbody,
                                     (zero,) * (2 * (D // 16)))
                for s2 in range(2):
                    for k in range(D // 16):
                        ostage_v[ob, half * 2 + s2, pl.ds(k * 16, 16)] = (
                            accs[s2 * (D // 16) + k])
            _ocopy(c, ob).start()

        for c in range(NBUF - 1):
            _gather(c, c).start()

        def _body(t, carry):
            c0 = t * NBUF
            for b in range(NBUF):
                _chunk(c0 + b, b, b % 2)
            return carry

        lax.fori_loop(0, nchunk // NBUF, _body, 0)
        _ocopy(nchunk - 2, 0).wait()
        _ocopy(nchunk - 1, 1).wait()

    mesh = plsc.VectorSubcoreMesh(core_axis_name="c", subcore_axis_name="s")
    return pl.kernel(
        _emb_sum_body,
        mesh=mesh,
        out_type=jax.ShapeDtypeStruct((nb, D), jnp.float32),
        scratch_types=[
            pltpu.VMEM((nchunk, IDX_PER_CHUNK), jnp.int32),
            pltpu.VMEM((NBUF, IDX_PER_CHUNK, D), jnp.float32),
            pltpu.VMEM((2, SAMP_PER_CHUNK, D), jnp.float32),
            pltpu.SemaphoreType.DMA((NBUF,)),
            pltpu.SemaphoreType.DMA((2,)),
        ],
    )


def _mlp_body(xn_ref, emb_ref, Wn_ref, bn_ref, W1_ref, b1_ref, W2_ref,
              b2_ref, Wo_ref, bo_ref, out_ref):
    x = (emb_ref[...]
         + jnp.dot(xn_ref[...], Wn_ref[...], preferred_element_type=jnp.float32)
         + bn_ref[...][None, :])
    for i in range(N_BLOCKS):
        y = jnp.maximum(
            jnp.dot(x, W1_ref[i], preferred_element_type=jnp.float32)
            + b1_ref[i][None, :], 0.0)
        y = jnp.maximum(
            jnp.dot(y, W2_ref[i], preferred_element_type=jnp.float32)
            + b2_ref[i][None, :], 0.0)
        x = x + y
    out_ref[...] = (jnp.dot(x, Wo_ref[...], preferred_element_type=jnp.float32)
                    + bo_ref[...][None, :])


def _mlp_call(xn, emb, Wn, bn, W1, b1, W2, b2, Wo, bo):
    nb = xn.shape[0]
    BM = 2048
    grid = (nb // BM,)
    row_spec = lambda w: pl.BlockSpec((BM, w), lambda i: (i, 0))
    full = lambda shape: pl.BlockSpec(shape, lambda i: tuple(0 for _ in shape))
    return pl.pallas_call(
        _mlp_body,
        grid=grid,
        in_specs=[
            row_spec(16), row_spec(D),
            full((16, D)), full((D,)),
            full((N_BLOCKS, D, D_HID)), full((N_BLOCKS, D_HID)),
            full((N_BLOCKS, D_HID, D)), full((N_BLOCKS, D)),
            full((D, 1)), full((1,)),
        ],
        out_specs=row_spec(1),
        out_shape=jax.ShapeDtypeStruct((nb, 1), jnp.float32),
    )(xn, emb, Wn, bn, W1, b1, W2, b2, Wo, bo)


def kernel(x_num, x_cat, W_num, b_num, tables, W1, b1, W2, b2, W_out, b_out):
    table_flat = tables.reshape(N_CAT * VOCAB, D)
    offs = (jnp.arange(N_CAT, dtype=jnp.int32) * VOCAB)[None, :]
    idx = x_cat + offs
    xn = jnp.pad(x_num, ((0, 0), (0, 16 - NUM_F)))
    Wn = jnp.pad(W_num, ((0, 16 - NUM_F), (0, 0)))

    nh = B // 2
    nchunk_h = nh // (NW * SAMP_PER_CHUNK)
    sc = _make_emb_sum_sc(nchunk_h)
    embs = [sc(table_flat,
               idx[h * nh:(h + 1) * nh].reshape(NW, nchunk_h, IDX_PER_CHUNK))
            for h in range(2)]
    outs = [_mlp_call(xn[h * nh:(h + 1) * nh], embs[h],
                      Wn, b_num, W1, b1, W2, b2, W_out, b_out)
            for h in range(2)]
    return jnp.concatenate(outs, axis=0)
